# baseline (speedup 1.0000x reference)
import functools

import jax
import jax.numpy as jnp
from jax.experimental import pallas as pl
from jax.experimental.pallas import tpu as pltpu


def _finalize(s, sq, g, b, count, eps):
    mean = s / count
    var = jnp.maximum(sq / count - mean * mean, 0.0)
    scale = g * jax.lax.rsqrt(var + eps)
    shift = b - mean * scale
    return scale, shift


def _conv_bn_stats_kernel(s_ref, sq_ref, g_ref, b_ref, x_ref, w_ref,
                          out_ref, sum_ref, sumsq_ref, xpad_ref, bn_ref,
                          *, apply_in_bn, count, eps):
    i = pl.program_id(0)

    h = x_ref.shape[1]
    w = x_ref.shape[2]
    cin = x_ref.shape[3]
    oh = out_ref.shape[1]
    ow = out_ref.shape[2]

    @pl.when(i == 0)
    def _():
        xpad_ref[0:1] = jnp.zeros_like(xpad_ref[0:1])
        xpad_ref[h + 1:h + 2] = jnp.zeros_like(xpad_ref[h + 1:h + 2])
        xpad_ref[:, 0:1, :] = jnp.zeros_like(xpad_ref[:, 0:1, :])
        xpad_ref[:, w + 1:w + 2, :] = jnp.zeros_like(xpad_ref[:, w + 1:w + 2, :])
        sum_ref[...] = jnp.zeros_like(sum_ref)
        sumsq_ref[...] = jnp.zeros_like(sumsq_ref)
        if apply_in_bn:
            scale, shift = _finalize(s_ref[...], sq_ref[...],
                                     g_ref[...], b_ref[...], count, eps)
            bn_ref[0:1] = scale
            bn_ref[1:2] = shift

    for b in range(x_ref.shape[0]):
        x = x_ref[b]
        if apply_in_bn:
            x = jnp.maximum(x.astype(jnp.float32) * bn_ref[0:1] + bn_ref[1:2],
                            0.0)
        xpad_ref[1:h + 1, 1:w + 1, :] = x

        taps = []
        for kh in range(3):
            for kw in range(3):
                taps.append(xpad_ref[kh:kh + oh, kw:kw + ow, :])
        patches = jnp.concatenate(taps, axis=-1).reshape(oh * ow, 9 * cin)

        acc = jnp.dot(patches, w_ref[...], preferred_element_type=jnp.float32)

        out_ref[b] = acc.reshape(oh, ow, -1).astype(out_ref.dtype)
        sum_ref[...] += jnp.sum(acc, axis=0, keepdims=True)
        sumsq_ref[...] += jnp.sum(acc * acc, axis=0, keepdims=True)


def _conv_bn_stats(x, w2d, s_in, sq_in, g_in, b_in, *, apply_in_bn, cout, eps, nb):
    n, h, w, cin = x.shape
    kfn = functools.partial(_conv_bn_stats_kernel, apply_in_bn=apply_in_bn,
                           count=float(n * h * w), eps=eps)
    return pl.pallas_call(
        kfn,
        grid=(n // nb,),
        in_specs=[
            pl.BlockSpec((1, cin), lambda i: (0, 0)),
            pl.BlockSpec((1, cin), lambda i: (0, 0)),
            pl.BlockSpec((1, cin), lambda i: (0, 0)),
            pl.BlockSpec((1, cin), lambda i: (0, 0)),
            pl.BlockSpec((nb, h, w, cin), lambda i: (i, 0, 0, 0)),
            pl.BlockSpec((9 * cin, cout), lambda i: (0, 0)),
        ],
        out_specs=[
            pl.BlockSpec((nb, h, w, cout), lambda i: (i, 0, 0, 0)),
            pl.BlockSpec((1, cout), lambda i: (0, 0)),
            pl.BlockSpec((1, cout), lambda i: (0, 0)),
        ],
        out_shape=(
            jax.ShapeDtypeStruct((n, h, w, cout), jnp.bfloat16),
            jax.ShapeDtypeStruct((1, cout), jnp.float32),
            jax.ShapeDtypeStruct((1, cout), jnp.float32),
        ),
        scratch_shapes=[
            pltpu.VMEM((h + 2, w + 2, cin), jnp.float32),
            pltpu.VMEM((2, cin), jnp.float32),
        ],
        compiler_params=pltpu.CompilerParams(
            dimension_semantics=("arbitrary",)),
    )(s_in, sq_in, g_in, b_in, x, w2d)


def _conv2_epi_kernel(s1_ref, sq1_ref, g1_ref, b1_ref, g2_ref, b2_ref,
                      x_ref, w_ref,
                      c2_ref, sum_ref, sumsq_ref, o_ref,
                      xpad_ref, bn1_ref, bn2_ref, slab_ref, rbuf_ref,
                      ssem, rsem, *, count, eps, n_cv):
    i = pl.program_id(0)
    nb = x_ref.shape[0]
    h = x_ref.shape[1]
    w = x_ref.shape[2]
    cin = x_ref.shape[3]
    oh, ow = h, w

    @pl.when(i == 0)
    def _():
        xpad_ref[0:1] = jnp.zeros_like(xpad_ref[0:1])
        xpad_ref[h + 1:h + 2] = jnp.zeros_like(xpad_ref[h + 1:h + 2])
        xpad_ref[:, 0:1, :] = jnp.zeros_like(xpad_ref[:, 0:1, :])
        xpad_ref[:, w + 1:w + 2, :] = jnp.zeros_like(xpad_ref[:, w + 1:w + 2, :])
        sum_ref[...] = jnp.zeros_like(sum_ref)
        sumsq_ref[...] = jnp.zeros_like(sumsq_ref)
        scale, shift = _finalize(s1_ref[...], sq1_ref[...],
                                 g1_ref[...], b1_ref[...], count, eps)
        bn1_ref[0:1] = scale
        bn1_ref[1:2] = shift

    slot = jax.lax.rem(i, 2)

    @pl.when(i < n_cv)
    def _conv_phase():
        @pl.when(i >= 2)
        def _():
            pltpu.make_async_copy(slab_ref.at[slot],
                                  c2_ref.at[pl.ds(i * nb, nb)],
                                  ssem.at[slot]).wait()
        for b in range(nb):
            x = jnp.maximum(
                x_ref[b].astype(jnp.float32) * bn1_ref[0:1] + bn1_ref[1:2],
                0.0)
            xpad_ref[1:h + 1, 1:w + 1, :] = x
            taps = []
            for kh in range(3):
                for kw in range(3):
                    taps.append(xpad_ref[kh:kh + oh, kw:kw + ow, :])
            patches = jnp.concatenate(taps, axis=-1).reshape(oh * ow, 9 * cin)
            acc = jnp.dot(patches, w_ref[...],
                          preferred_element_type=jnp.float32)
            slab_ref[slot, b] = acc.reshape(oh, ow, -1).astype(slab_ref.dtype)
            sum_ref[...] += jnp.sum(acc, axis=0, keepdims=True)
            sumsq_ref[...] += jnp.sum(acc * acc, axis=0, keepdims=True)
        pltpu.make_async_copy(slab_ref.at[slot],
                              c2_ref.at[pl.ds(i * nb, nb)],
                              ssem.at[slot]).start()
        @pl.when(i == n_cv - 1)
        def _():
            pltpu.make_async_copy(c2_ref.at[pl.ds(0, nb)],
                                  rbuf_ref.at[0], rsem.at[0]).start()

    @pl.when(i == n_cv)
    def _drain():
        pltpu.make_async_copy(slab_ref.at[0],
                              c2_ref.at[pl.ds((n_cv - 2) * nb, nb)],
                              ssem.at[0]).wait()
        pltpu.make_async_copy(slab_ref.at[1],
                              c2_ref.at[pl.ds((n_cv - 1) * nb, nb)],
                              ssem.at[1]).wait()
        scale, shift = _finalize(sum_ref[...], sumsq_ref[...],
                                 g2_ref[...], b2_ref[...], count, eps)
        bn2_ref[0:1] = scale
        bn2_ref[1:2] = shift

    @pl.when(i >= n_cv)
    def _epi_phase():
        j = i - n_cv
        eslot = jax.lax.rem(j, 2)
        pltpu.make_async_copy(c2_ref.at[pl.ds(j * nb, nb)],
                              rbuf_ref.at[eslot], rsem.at[eslot]).wait()
        @pl.when(j < n_cv - 1)
        def _():
            pltpu.make_async_copy(c2_ref.at[pl.ds((j + 1) * nb, nb)],
                                  rbuf_ref.at[1 - eslot],
                                  rsem.at[1 - eslot]).start()
        xb = rbuf_ref[eslot].astype(jnp.float32)
        o_ref[...] = jnp.maximum(xb * bn2_ref[0:1] + bn2_ref[1:2], 0.0)


def _conv2_epilogue(conv1, w2d, s1, sq1, g1, b1, g2, b2, *, eps):
    n, h, w, cin = conv1.shape
    cout = w2d.shape[-1]
    nb = 4
    n_cv = n // nb
    kfn = functools.partial(_conv2_epi_kernel,
                           count=float(n * h * w), eps=eps, n_cv=n_cv)
    _, _, _, out = pl.pallas_call(
        kfn,
        grid=(2 * n_cv,),
        in_specs=[
            pl.BlockSpec((1, cin), lambda i: (0, 0)),
            pl.BlockSpec((1, cin), lambda i: (0, 0)),
            pl.BlockSpec((1, cin), lambda i: (0, 0)),
            pl.BlockSpec((1, cin), lambda i: (0, 0)),
            pl.BlockSpec((1, cout), lambda i: (0, 0)),
            pl.BlockSpec((1, cout), lambda i: (0, 0)),
            pl.BlockSpec((nb, h, w, cin),
                         lambda i: (jnp.minimum(i, n_cv - 1), 0, 0, 0)),
            pl.BlockSpec((9 * cin, cout), lambda i: (0, 0)),
        ],
        out_specs=[
            pl.BlockSpec(memory_space=pl.ANY),
            pl.BlockSpec((1, cout), lambda i: (0, 0)),
            pl.BlockSpec((1, cout), lambda i: (0, 0)),
            pl.BlockSpec((nb, h, w, cout),
                         lambda i: (jnp.maximum(i - n_cv, 0), 0, 0, 0)),
        ],
        out_shape=(
            jax.ShapeDtypeStruct((n, h, w, cout), jnp.bfloat16),
            jax.ShapeDtypeStruct((1, cout), jnp.float32),
            jax.ShapeDtypeStruct((1, cout), jnp.float32),
            jax.ShapeDtypeStruct((n, h, w, cout), jnp.float32),
        ),
        scratch_shapes=[
            pltpu.VMEM((h + 2, w + 2, cin), jnp.float32),
            pltpu.VMEM((2, cin), jnp.float32),
            pltpu.VMEM((2, cout), jnp.float32),
            pltpu.VMEM((2, nb, h, w, cout), jnp.bfloat16),
            pltpu.VMEM((2, nb, h, w, cout), jnp.bfloat16),
            pltpu.SemaphoreType.DMA((2,)),
            pltpu.SemaphoreType.DMA((2,)),
        ],
        compiler_params=pltpu.CompilerParams(
            dimension_semantics=("arbitrary",)),
    )(s1, sq1, g1.reshape(1, -1), b1.reshape(1, -1),
      g2.reshape(1, -1), b2.reshape(1, -1), conv1, w2d)
    return out


def _bn_relu_kernel(s_ref, sq_ref, g_ref, b_ref, x_ref, o_ref, bn_ref,
                    *, count, eps):
    @pl.when(pl.program_id(0) == 0)
    def _():
        scale, shift = _finalize(s_ref[...], sq_ref[...],
                                 g_ref[...], b_ref[...], count, eps)
        bn_ref[0:1] = scale
        bn_ref[1:2] = shift

    xf = x_ref[...].astype(jnp.float32)
    o_ref[...] = jnp.maximum(xf * bn_ref[0:1] + bn_ref[1:2], 0.0)


def _bn_relu(x, s_in, sq_in, g_in, b_in, nb, eps):
    n, h, w, c = x.shape
    kfn = functools.partial(_bn_relu_kernel, count=float(n * h * w), eps=eps)
    return pl.pallas_call(
        kfn,
        grid=(n // nb,),
        in_specs=[
            pl.BlockSpec((1, c), lambda i: (0, 0)),
            pl.BlockSpec((1, c), lambda i: (0, 0)),
            pl.BlockSpec((1, c), lambda i: (0, 0)),
            pl.BlockSpec((1, c), lambda i: (0, 0)),
            pl.BlockSpec((nb, h, w, c), lambda i: (i, 0, 0, 0)),
        ],
        out_specs=pl.BlockSpec((nb, h, w, c), lambda i: (i, 0, 0, 0)),
        out_shape=jax.ShapeDtypeStruct((n, h, w, c), jnp.float32),
        scratch_shapes=[pltpu.VMEM((2, c), jnp.float32)],
        compiler_params=pltpu.CompilerParams(
            dimension_semantics=("arbitrary",)),
    )(s_in, sq_in, g_in, b_in, x)


def kernel(x_nhwc, w1, w2, g1, b1, g2, b2, *, eps=1e-5):
    n, h, w, cin = x_nhwc.shape
    cout1 = w1.shape[-1]
    cout2 = w2.shape[-1]
    w1_2d = w1.reshape(9 * cin, cout1)
    w2_2d = w2.reshape(9 * cout1, cout2)

    ones = jnp.ones((1, cin), jnp.float32)
    zeros = jnp.zeros((1, cin), jnp.float32)

    conv1, s1, sq1 = _conv_bn_stats(x_nhwc, w1_2d, ones, ones, ones, zeros,
                                    apply_in_bn=False, cout=cout1, eps=eps, nb=2)

    return _conv2_epilogue(conv1, w2_2d, s1, sq1, g1, b1, g2, b2, eps=eps)


# speedup vs baseline: 1.0177x; 1.0177x over previous
import functools

import jax
import jax.numpy as jnp
from jax.experimental import pallas as pl
from jax.experimental.pallas import tpu as pltpu


def _finalize(s, sq, g, b, count, eps):
    mean = s / count
    var = jnp.maximum(sq / count - mean * mean, 0.0)
    scale = g * jax.lax.rsqrt(var + eps)
    shift = b - mean * scale
    return scale, shift


def _conv_bn_stats_kernel(s_ref, sq_ref, g_ref, b_ref, x_ref, w_ref,
                          out_ref, sum_ref, sumsq_ref, xpad_ref, bn_ref,
                          *, apply_in_bn, count, eps):
    i = pl.program_id(0)

    h = x_ref.shape[1]
    w = x_ref.shape[2]
    cin = x_ref.shape[3]
    oh = out_ref.shape[1]
    ow = out_ref.shape[2]

    @pl.when(i == 0)
    def _():
        xpad_ref[0:1] = jnp.zeros_like(xpad_ref[0:1])
        xpad_ref[h + 1:h + 2] = jnp.zeros_like(xpad_ref[h + 1:h + 2])
        xpad_ref[:, 0:1, :] = jnp.zeros_like(xpad_ref[:, 0:1, :])
        xpad_ref[:, w + 1:w + 2, :] = jnp.zeros_like(xpad_ref[:, w + 1:w + 2, :])
        sum_ref[...] = jnp.zeros_like(sum_ref)
        sumsq_ref[...] = jnp.zeros_like(sumsq_ref)
        if apply_in_bn:
            scale, shift = _finalize(s_ref[...], sq_ref[...],
                                     g_ref[...], b_ref[...], count, eps)
            bn_ref[0:1] = scale
            bn_ref[1:2] = shift

    for b in range(x_ref.shape[0]):
        x = x_ref[b]
        if apply_in_bn:
            x = jnp.maximum(x.astype(jnp.float32) * bn_ref[0:1] + bn_ref[1:2],
                            0.0)
        xpad_ref[1:h + 1, 1:w + 1, :] = x

        taps = []
        for kh in range(3):
            for kw in range(3):
                taps.append(xpad_ref[kh:kh + oh, kw:kw + ow, :])
        patches = jnp.concatenate(taps, axis=-1).reshape(oh * ow, 9 * cin)

        acc = jnp.dot(patches, w_ref[...], preferred_element_type=jnp.float32)

        out_ref[b] = acc.reshape(oh, ow, -1).astype(out_ref.dtype)
        sum_ref[...] += jnp.sum(acc, axis=0, keepdims=True)
        sumsq_ref[...] += jnp.sum(acc * acc, axis=0, keepdims=True)


def _conv_bn_stats(x, w2d, s_in, sq_in, g_in, b_in, *, apply_in_bn, cout, eps, nb):
    n, h, w, cin = x.shape
    kfn = functools.partial(_conv_bn_stats_kernel, apply_in_bn=apply_in_bn,
                           count=float(n * h * w), eps=eps)
    return pl.pallas_call(
        kfn,
        grid=(n // nb,),
        in_specs=[
            pl.BlockSpec((1, cin), lambda i: (0, 0)),
            pl.BlockSpec((1, cin), lambda i: (0, 0)),
            pl.BlockSpec((1, cin), lambda i: (0, 0)),
            pl.BlockSpec((1, cin), lambda i: (0, 0)),
            pl.BlockSpec((nb, h, w, cin), lambda i: (i, 0, 0, 0)),
            pl.BlockSpec((9 * cin, cout), lambda i: (0, 0)),
        ],
        out_specs=[
            pl.BlockSpec((nb, h, w, cout), lambda i: (i, 0, 0, 0)),
            pl.BlockSpec((1, cout), lambda i: (0, 0)),
            pl.BlockSpec((1, cout), lambda i: (0, 0)),
        ],
        out_shape=(
            jax.ShapeDtypeStruct((n, h, w, cout), jnp.bfloat16),
            jax.ShapeDtypeStruct((1, cout), jnp.float32),
            jax.ShapeDtypeStruct((1, cout), jnp.float32),
        ),
        scratch_shapes=[
            pltpu.VMEM((h + 2, w + 2, cin), jnp.float32),
            pltpu.VMEM((2, cin), jnp.float32),
        ],
        compiler_params=pltpu.CompilerParams(
            dimension_semantics=("arbitrary",)),
    )(s_in, sq_in, g_in, b_in, x, w2d)


def _bn_relu_kernel(s_ref, sq_ref, g_ref, b_ref, x_ref, o_ref, bn_ref,
                    *, count, eps):
    @pl.when(pl.program_id(0) == 0)
    def _():
        scale, shift = _finalize(s_ref[...], sq_ref[...],
                                 g_ref[...], b_ref[...], count, eps)
        bn_ref[0:1] = scale
        bn_ref[1:2] = shift

    xf = x_ref[...].astype(jnp.float32)
    o_ref[...] = jnp.maximum(xf * bn_ref[0:1] + bn_ref[1:2], 0.0)


def _bn_relu(x, s_in, sq_in, g_in, b_in, nb, eps):
    n, h, w, c = x.shape
    kfn = functools.partial(_bn_relu_kernel, count=float(n * h * w), eps=eps)
    return pl.pallas_call(
        kfn,
        grid=(n // nb,),
        in_specs=[
            pl.BlockSpec((1, c), lambda i: (0, 0)),
            pl.BlockSpec((1, c), lambda i: (0, 0)),
            pl.BlockSpec((1, c), lambda i: (0, 0)),
            pl.BlockSpec((1, c), lambda i: (0, 0)),
            pl.BlockSpec((nb, h, w, c), lambda i: (i, 0, 0, 0)),
        ],
        out_specs=pl.BlockSpec((nb, h, w, c), lambda i: (i, 0, 0, 0)),
        out_shape=jax.ShapeDtypeStruct((n, h, w, c), jnp.float32),
        scratch_shapes=[pltpu.VMEM((2, c), jnp.float32)],
        compiler_params=pltpu.CompilerParams(
            dimension_semantics=("arbitrary",)),
    )(s_in, sq_in, g_in, b_in, x)


def kernel(x_nhwc, w1, w2, g1, b1, g2, b2, *, eps=1e-5):
    n, h, w, cin = x_nhwc.shape
    cout1 = w1.shape[-1]
    cout2 = w2.shape[-1]
    w1_2d = w1.reshape(9 * cin, cout1)
    w2_2d = w2.reshape(9 * cout1, cout2)

    ones = jnp.ones((1, cin), jnp.float32)
    zeros = jnp.zeros((1, cin), jnp.float32)

    conv1, s1, sq1 = _conv_bn_stats(x_nhwc, w1_2d, ones, ones, ones, zeros,
                                    apply_in_bn=False, cout=cout1, eps=eps, nb=2)

    conv2, s2, sq2 = _conv_bn_stats(conv1, w2_2d, s1, sq1,
                                    g1.reshape(1, -1), b1.reshape(1, -1),
                                    apply_in_bn=True, cout=cout2, eps=eps, nb=4)

    return _bn_relu(conv2, s2, sq2, g2.reshape(1, -1), b2.reshape(1, -1),
                    nb=8, eps=eps)
